# baseline (speedup 1.0000x reference)
import functools

import jax
import jax.numpy as jnp
from jax import lax
from jax.experimental import pallas as pl
from jax.experimental.pallas import tpu as pltpu
from jax.experimental.pallas import tpu_sc as plsc

_NUM_CORES = 2
_NUM_SUBCORES = 16
_NUM_WORKERS = _NUM_CORES * _NUM_SUBCORES
_LANES = 16
_CH = 2048


def _chunk_plan(e):
    assert e % _CH == 0, f"edge count {e} not divisible by {_CH}"
    nch_tot = e // _CH
    nfull = nch_tot // _NUM_WORKERS
    rem = nch_tot - nfull * _NUM_WORKERS
    assert nfull >= 2
    maxc = nfull + (1 if rem else 0)
    return nch_tot, nfull, rem, maxc


def _worker_id():
    return lax.axis_index("s") * _NUM_CORES + lax.axis_index("c")


def _scatter_partials(edge_index, values, np_pad):
    e = values.shape[0]
    packed = edge_index.ndim == 2
    nch_tot, nfull, rem, maxc = _chunk_plan(e)
    gp = _CH // _LANES
    ibuf_t = (pltpu.VMEM((2, _CH), jnp.int32) if packed
              else pltpu.VMEM((_CH,), jnp.int32))

    mesh = plsc.VectorSubcoreMesh(core_axis_name="c", subcore_axis_name="s")

    @functools.partial(
        pl.kernel,
        out_type=jax.ShapeDtypeStruct((_NUM_WORKERS, 1, np_pad), jnp.float32),
        mesh=mesh,
        compiler_params=pltpu.CompilerParams(needs_layout_passes=False),
        scratch_types=[
            pltpu.VMEM((np_pad,), jnp.float32),
            ibuf_t,
            ibuf_t,
            pltpu.VMEM((_CH,), jnp.float32),
            pltpu.VMEM((_CH,), jnp.float32),
            pltpu.SemaphoreType.DMA,
            pltpu.SemaphoreType.DMA,
            pltpu.SemaphoreType.DMA,
            pltpu.SemaphoreType.DMA,
        ],
    )
    def scatter_k(ei_hbm, val_hbm, out_hbm, acc, ib0, ib1, vb0, vb1,
                  si0, si1, sv0, sv1):
        wid = _worker_id()
        base_chunk = wid * nfull + jnp.minimum(wid, rem)
        count = nfull + jnp.where(wid < rem, 1, 0)

        zeros = jnp.zeros((_LANES,), jnp.float32)

        def zero_body(i, carry):
            acc[pl.ds(i * _LANES, _LANES)] = zeros
            return carry

        lax.fori_loop(0, np_pad // _LANES, zero_body, 0, unroll=8)

        def start(ci, ib, vb, si, sv):
            off = (base_chunk + ci) * _CH
            src = (ei_hbm.at[:, pl.ds(off, _CH)] if packed
                   else ei_hbm.at[pl.ds(off, _CH)])
            pltpu.async_copy(src, ib, si)
            pltpu.async_copy(val_hbm.at[pl.ds(off, _CH)], vb, sv)

        def wait(ib, vb, si, sv):
            src = (ei_hbm.at[:, pl.ds(0, _CH)] if packed
                   else ei_hbm.at[pl.ds(0, _CH)])
            pltpu.make_async_copy(src, ib, si).wait()
            pltpu.make_async_copy(val_hbm.at[pl.ds(0, _CH)], vb, sv).wait()

        def process(ib, vb):
            def body(j, carry):
                if packed:
                    idx = ib[1, pl.ds(j * _LANES, _LANES)]
                else:
                    idx = ib[pl.ds(j * _LANES, _LANES)]
                v = vb[pl.ds(j * _LANES, _LANES)]
                plsc.addupdate_scatter(acc, [idx], v)
                return carry

            lax.fori_loop(0, gp, body, 0, unroll=16)

        start(0, ib0, vb0, si0, sv0)
        start(1, ib1, vb1, si1, sv1)

        def pair(p, carry):
            ci0 = p * 2
            ci1 = ci0 + 1

            @pl.when(ci0 < count)
            def _():
                wait(ib0, vb0, si0, sv0)
                process(ib0, vb0)

            @pl.when(ci0 + 2 < count)
            def _():
                start(ci0 + 2, ib0, vb0, si0, sv0)

            @pl.when(ci1 < count)
            def _():
                wait(ib1, vb1, si1, sv1)
                process(ib1, vb1)

            @pl.when(ci1 + 2 < count)
            def _():
                start(ci1 + 2, ib1, vb1, si1, sv1)

            return carry

        lax.fori_loop(0, (maxc + 1) // 2, pair, 0)
        pltpu.sync_copy(acc, out_hbm.at[wid, 0, pl.ds(0, np_pad)])

    return scatter_k(edge_index, values)


def _gather_mul(edge_index, values, s_flat):
    e = values.shape[0]
    n_pad = s_flat.shape[0]
    nch_tot, nfull, rem, maxc = _chunk_plan(e)
    gp = _CH // _LANES

    mesh = plsc.VectorSubcoreMesh(core_axis_name="c", subcore_axis_name="s")

    @functools.partial(
        pl.kernel,
        out_type=jax.ShapeDtypeStruct((e,), jnp.float32),
        mesh=mesh,
        compiler_params=pltpu.CompilerParams(needs_layout_passes=False),
        scratch_types=[
            pltpu.VMEM((n_pad,), jnp.float32),
            pltpu.VMEM((_CH,), jnp.int32),
            pltpu.VMEM((_CH,), jnp.int32),
            pltpu.VMEM((_CH,), jnp.float32),
            pltpu.VMEM((_CH,), jnp.float32),
            pltpu.VMEM((_CH,), jnp.float32),
            pltpu.VMEM((_CH,), jnp.float32),
            pltpu.SemaphoreType.DMA,
            pltpu.SemaphoreType.DMA,
            pltpu.SemaphoreType.DMA,
            pltpu.SemaphoreType.DMA,
            pltpu.SemaphoreType.DMA,
            pltpu.SemaphoreType.DMA,
            pltpu.SemaphoreType.DMA,
        ],
    )
    def gather_k(ei_hbm, val_hbm, s_hbm, m_hbm, sv, ib0, ib1, wb0, wb1,
                 mb0, mb1, stab, si0, si1, sw0, sw1, so0, so1):
        wid = _worker_id()
        base_chunk = wid * nfull + jnp.minimum(wid, rem)
        count = nfull + jnp.where(wid < rem, 1, 0)

        pltpu.async_copy(s_hbm, sv, stab).wait()

        def start_in(ci, ib, wb, si, sw):
            off = (base_chunk + ci) * _CH
            pltpu.async_copy(ei_hbm.at[0, pl.ds(off, _CH)], ib, si)
            pltpu.async_copy(val_hbm.at[pl.ds(off, _CH)], wb, sw)

        def wait_in(ib, wb, si, sw):
            pltpu.make_async_copy(ei_hbm.at[0, pl.ds(0, _CH)], ib, si).wait()
            pltpu.make_async_copy(val_hbm.at[pl.ds(0, _CH)], wb, sw).wait()

        def wait_out(mb, so):
            pltpu.make_async_copy(mb, m_hbm.at[pl.ds(0, _CH)], so).wait()

        def process(ib, wb, mb):
            def body(j, carry):
                idx = ib[pl.ds(j * _LANES, _LANES)]
                w = wb[pl.ds(j * _LANES, _LANES)]
                vals = plsc.load_gather(sv, [idx])
                mb[pl.ds(j * _LANES, _LANES)] = vals * w
                return carry

            lax.fori_loop(0, gp, body, 0, unroll=16)

        def start_out(ci, mb, so):
            off = (base_chunk + ci) * _CH
            pltpu.async_copy(mb, m_hbm.at[pl.ds(off, _CH)], so)

        start_in(0, ib0, wb0, si0, sw0)
        start_in(1, ib1, wb1, si1, sw1)

        def pair(p, carry):
            ci0 = p * 2
            ci1 = ci0 + 1

            @pl.when(ci0 < count)
            def _():
                wait_in(ib0, wb0, si0, sw0)

                @pl.when(p > 0)
                def _():
                    wait_out(mb0, so0)

                process(ib0, wb0, mb0)
                start_out(ci0, mb0, so0)

            @pl.when(ci0 + 2 < count)
            def _():
                start_in(ci0 + 2, ib0, wb0, si0, sw0)

            @pl.when(ci1 < count)
            def _():
                wait_in(ib1, wb1, si1, sw1)

                @pl.when(p > 0)
                def _():
                    wait_out(mb1, so1)

                process(ib1, wb1, mb1)
                start_out(ci1, mb1, so1)

            @pl.when(ci1 + 2 < count)
            def _():
                start_in(ci1 + 2, ib1, wb1, si1, sw1)

            return carry

        lax.fori_loop(0, (maxc + 1) // 2, pair, 0)
        wait_out(mb0, so0)

        @pl.when(count > 1)
        def _():
            wait_out(mb1, so1)

    return gather_k(edge_index, values, s_flat)


def _dis_s(degp, x_row, np_pad, bn=8192):
    def body(degp_ref, x_ref, dis_ref, s_ref):
        d = jnp.sum(degp_ref[...], axis=0)
        dis = jnp.where(d > 0.0, lax.rsqrt(jnp.maximum(d, 1e-12)), 0.0)
        dis_ref[...] = dis
        s_ref[...] = dis * x_ref[...]

    grid = pl.cdiv(np_pad, bn)
    return pl.pallas_call(
        body,
        grid=(grid,),
        in_specs=[
            pl.BlockSpec((_NUM_WORKERS, 1, bn), lambda i: (0, 0, i)),
            pl.BlockSpec((1, bn), lambda i: (0, i)),
        ],
        out_specs=[
            pl.BlockSpec((1, bn), lambda i: (0, i)),
            pl.BlockSpec((1, bn), lambda i: (0, i)),
        ],
        out_shape=[
            jax.ShapeDtypeStruct((1, np_pad), jnp.float32),
            jax.ShapeDtypeStruct((1, np_pad), jnp.float32),
        ],
    )(degp, x_row)


def _tx1_dense(accp, dis_row, x_row, w0c, w1c, bc, n_nodes, out_c, bn=2048):
    def body(accp_ref, dis_ref, x_ref, w0c_ref, w1c_ref, bc_ref, o_ref):
        a = jnp.sum(accp_ref[...], axis=0)
        t = -dis_ref[...] * a
        ot = (x_ref[...] * w0c_ref[...] + t * w1c_ref[...]
              + bc_ref[...])
        ot = jnp.where(ot >= 0.0, ot, 0.01 * ot)
        o_ref[...] = ot.T

    grid = pl.cdiv(n_nodes, bn)
    return pl.pallas_call(
        body,
        grid=(grid,),
        in_specs=[
            pl.BlockSpec((_NUM_WORKERS, 1, bn), lambda i: (0, 0, i)),
            pl.BlockSpec((1, bn), lambda i: (0, i)),
            pl.BlockSpec((1, bn), lambda i: (0, i)),
            pl.BlockSpec((out_c, 1), lambda i: (0, 0)),
            pl.BlockSpec((out_c, 1), lambda i: (0, 0)),
            pl.BlockSpec((out_c, 1), lambda i: (0, 0)),
        ],
        out_specs=pl.BlockSpec((bn, out_c), lambda i: (i, 0)),
        out_shape=jax.ShapeDtypeStruct((n_nodes, out_c), jnp.float32),
    )(accp, dis_row, x_row, w0c, w1c, bc)


def kernel(x, edge_index, edge_weight, W0, W1, b):
    n = x.shape[0]
    out_c = W0.shape[1]
    np_pad = ((n + 127) // 128) * 128
    ei = edge_index.astype(jnp.int32)
    ew = edge_weight.astype(jnp.float32)

    degp = _scatter_partials(ei, ew, np_pad)
    x_row = x.reshape(1, n)
    dis_row, s_row = _dis_s(degp, x_row, np_pad)
    dst_flat = ei[1]
    m = _gather_mul(ei, ew, s_row.reshape(np_pad))
    accp = _scatter_partials(dst_flat, m, np_pad)
    w0c = W0.astype(jnp.float32).reshape(out_c, 1)
    w1c = W1.astype(jnp.float32).reshape(out_c, 1)
    bc = b.astype(jnp.float32).reshape(out_c, 1)
    out = _tx1_dense(accp, dis_row, x_row, w0c, w1c, bc, n, out_c)
    return out


# speedup vs baseline: 1.0009x; 1.0009x over previous
import functools

import jax
import jax.numpy as jnp
from jax import lax
from jax.experimental import pallas as pl
from jax.experimental.pallas import tpu as pltpu
from jax.experimental.pallas import tpu_sc as plsc

_NUM_CORES = 2
_NUM_SUBCORES = 16
_NUM_WORKERS = _NUM_CORES * _NUM_SUBCORES
_LANES = 16
_CH = 2048


def _chunk_plan(e):
    assert e % _CH == 0, f"edge count {e} not divisible by {_CH}"
    nch_tot = e // _CH
    nfull = nch_tot // _NUM_WORKERS
    rem = nch_tot - nfull * _NUM_WORKERS
    assert nfull >= 2
    maxc = nfull + (1 if rem else 0)
    return nch_tot, nfull, rem, maxc


def _worker_id():
    return lax.axis_index("s") * _NUM_CORES + lax.axis_index("c")


def _scatter_partials(edge_index, values, np_pad):
    e = values.shape[0]
    packed = edge_index.ndim == 2
    nch_tot, nfull, rem, maxc = _chunk_plan(e)
    gp = _CH // _LANES
    ibuf_t = (pltpu.VMEM((2, _CH), jnp.int32) if packed
              else pltpu.VMEM((_CH,), jnp.int32))

    mesh = plsc.VectorSubcoreMesh(core_axis_name="c", subcore_axis_name="s")

    @functools.partial(
        pl.kernel,
        out_type=jax.ShapeDtypeStruct((_NUM_WORKERS, 1, np_pad), jnp.float32),
        mesh=mesh,
        compiler_params=pltpu.CompilerParams(needs_layout_passes=False),
        scratch_types=[
            pltpu.VMEM((np_pad,), jnp.float32),
            ibuf_t,
            ibuf_t,
            pltpu.VMEM((_CH,), jnp.float32),
            pltpu.VMEM((_CH,), jnp.float32),
            pltpu.SemaphoreType.DMA,
            pltpu.SemaphoreType.DMA,
            pltpu.SemaphoreType.DMA,
            pltpu.SemaphoreType.DMA,
        ],
    )
    def scatter_k(ei_hbm, val_hbm, out_hbm, acc, ib0, ib1, vb0, vb1,
                  si0, si1, sv0, sv1):
        wid = _worker_id()
        base_chunk = wid * nfull + jnp.minimum(wid, rem)
        count = nfull + jnp.where(wid < rem, 1, 0)

        zeros = jnp.zeros((_LANES,), jnp.float32)

        def zero_body(i, carry):
            acc[pl.ds(i * _LANES, _LANES)] = zeros
            return carry

        lax.fori_loop(0, np_pad // _LANES, zero_body, 0, unroll=8)

        def start(ci, ib, vb, si, sv):
            off = (base_chunk + ci) * _CH
            src = (ei_hbm.at[:, pl.ds(off, _CH)] if packed
                   else ei_hbm.at[pl.ds(off, _CH)])
            pltpu.async_copy(src, ib, si)
            pltpu.async_copy(val_hbm.at[pl.ds(off, _CH)], vb, sv)

        def wait(ib, vb, si, sv):
            src = (ei_hbm.at[:, pl.ds(0, _CH)] if packed
                   else ei_hbm.at[pl.ds(0, _CH)])
            pltpu.make_async_copy(src, ib, si).wait()
            pltpu.make_async_copy(val_hbm.at[pl.ds(0, _CH)], vb, sv).wait()

        def process(ib, vb):
            def body(j, carry):
                if packed:
                    idx = ib[1, pl.ds(j * _LANES, _LANES)]
                else:
                    idx = ib[pl.ds(j * _LANES, _LANES)]
                v = vb[pl.ds(j * _LANES, _LANES)]
                plsc.addupdate_scatter(acc, [idx], v)
                return carry

            lax.fori_loop(0, gp, body, 0, unroll=8)

        start(0, ib0, vb0, si0, sv0)
        start(1, ib1, vb1, si1, sv1)

        def pair(p, carry):
            ci0 = p * 2
            ci1 = ci0 + 1

            @pl.when(ci0 < count)
            def _():
                wait(ib0, vb0, si0, sv0)
                process(ib0, vb0)

            @pl.when(ci0 + 2 < count)
            def _():
                start(ci0 + 2, ib0, vb0, si0, sv0)

            @pl.when(ci1 < count)
            def _():
                wait(ib1, vb1, si1, sv1)
                process(ib1, vb1)

            @pl.when(ci1 + 2 < count)
            def _():
                start(ci1 + 2, ib1, vb1, si1, sv1)

            return carry

        lax.fori_loop(0, (maxc + 1) // 2, pair, 0)
        pltpu.sync_copy(acc, out_hbm.at[wid, 0, pl.ds(0, np_pad)])

    return scatter_k(edge_index, values)


def _gather_mul(edge_index, values, s_flat):
    e = values.shape[0]
    n_pad = s_flat.shape[0]
    nch_tot, nfull, rem, maxc = _chunk_plan(e)
    gp = _CH // _LANES

    mesh = plsc.VectorSubcoreMesh(core_axis_name="c", subcore_axis_name="s")

    @functools.partial(
        pl.kernel,
        out_type=jax.ShapeDtypeStruct((e,), jnp.float32),
        mesh=mesh,
        compiler_params=pltpu.CompilerParams(needs_layout_passes=False),
        scratch_types=[
            pltpu.VMEM((n_pad,), jnp.float32),
            pltpu.VMEM((_CH,), jnp.int32),
            pltpu.VMEM((_CH,), jnp.int32),
            pltpu.VMEM((_CH,), jnp.float32),
            pltpu.VMEM((_CH,), jnp.float32),
            pltpu.VMEM((_CH,), jnp.float32),
            pltpu.VMEM((_CH,), jnp.float32),
            pltpu.SemaphoreType.DMA,
            pltpu.SemaphoreType.DMA,
            pltpu.SemaphoreType.DMA,
            pltpu.SemaphoreType.DMA,
            pltpu.SemaphoreType.DMA,
            pltpu.SemaphoreType.DMA,
            pltpu.SemaphoreType.DMA,
        ],
    )
    def gather_k(ei_hbm, val_hbm, s_hbm, m_hbm, sv, ib0, ib1, wb0, wb1,
                 mb0, mb1, stab, si0, si1, sw0, sw1, so0, so1):
        wid = _worker_id()
        base_chunk = wid * nfull + jnp.minimum(wid, rem)
        count = nfull + jnp.where(wid < rem, 1, 0)

        pltpu.async_copy(s_hbm, sv, stab).wait()

        def start_in(ci, ib, wb, si, sw):
            off = (base_chunk + ci) * _CH
            pltpu.async_copy(ei_hbm.at[0, pl.ds(off, _CH)], ib, si)
            pltpu.async_copy(val_hbm.at[pl.ds(off, _CH)], wb, sw)

        def wait_in(ib, wb, si, sw):
            pltpu.make_async_copy(ei_hbm.at[0, pl.ds(0, _CH)], ib, si).wait()
            pltpu.make_async_copy(val_hbm.at[pl.ds(0, _CH)], wb, sw).wait()

        def wait_out(mb, so):
            pltpu.make_async_copy(mb, m_hbm.at[pl.ds(0, _CH)], so).wait()

        def process(ib, wb, mb):
            def body(j, carry):
                idx = ib[pl.ds(j * _LANES, _LANES)]
                w = wb[pl.ds(j * _LANES, _LANES)]
                vals = plsc.load_gather(sv, [idx])
                mb[pl.ds(j * _LANES, _LANES)] = vals * w
                return carry

            lax.fori_loop(0, gp, body, 0, unroll=8)

        def start_out(ci, mb, so):
            off = (base_chunk + ci) * _CH
            pltpu.async_copy(mb, m_hbm.at[pl.ds(off, _CH)], so)

        start_in(0, ib0, wb0, si0, sw0)
        start_in(1, ib1, wb1, si1, sw1)

        def pair(p, carry):
            ci0 = p * 2
            ci1 = ci0 + 1

            @pl.when(ci0 < count)
            def _():
                wait_in(ib0, wb0, si0, sw0)

                @pl.when(p > 0)
                def _():
                    wait_out(mb0, so0)

                process(ib0, wb0, mb0)
                start_out(ci0, mb0, so0)

            @pl.when(ci0 + 2 < count)
            def _():
                start_in(ci0 + 2, ib0, wb0, si0, sw0)

            @pl.when(ci1 < count)
            def _():
                wait_in(ib1, wb1, si1, sw1)

                @pl.when(p > 0)
                def _():
                    wait_out(mb1, so1)

                process(ib1, wb1, mb1)
                start_out(ci1, mb1, so1)

            @pl.when(ci1 + 2 < count)
            def _():
                start_in(ci1 + 2, ib1, wb1, si1, sw1)

            return carry

        lax.fori_loop(0, (maxc + 1) // 2, pair, 0)
        wait_out(mb0, so0)

        @pl.when(count > 1)
        def _():
            wait_out(mb1, so1)

    return gather_k(edge_index, values, s_flat)


def _dis_s(degp, x_row, np_pad, bn=8192):
    def body(degp_ref, x_ref, dis_ref, s_ref):
        d = jnp.sum(degp_ref[...], axis=0)
        dis = jnp.where(d > 0.0, lax.rsqrt(jnp.maximum(d, 1e-12)), 0.0)
        dis_ref[...] = dis
        s_ref[...] = dis * x_ref[...]

    grid = pl.cdiv(np_pad, bn)
    return pl.pallas_call(
        body,
        grid=(grid,),
        in_specs=[
            pl.BlockSpec((_NUM_WORKERS, 1, bn), lambda i: (0, 0, i)),
            pl.BlockSpec((1, bn), lambda i: (0, i)),
        ],
        out_specs=[
            pl.BlockSpec((1, bn), lambda i: (0, i)),
            pl.BlockSpec((1, bn), lambda i: (0, i)),
        ],
        out_shape=[
            jax.ShapeDtypeStruct((1, np_pad), jnp.float32),
            jax.ShapeDtypeStruct((1, np_pad), jnp.float32),
        ],
    )(degp, x_row)


def _tx1_dense(accp, dis_row, x_row, w0c, w1c, bc, n_nodes, out_c, bn=2048):
    def body(accp_ref, dis_ref, x_ref, w0c_ref, w1c_ref, bc_ref, o_ref):
        a = jnp.sum(accp_ref[...], axis=0)
        t = -dis_ref[...] * a
        ot = (x_ref[...] * w0c_ref[...] + t * w1c_ref[...]
              + bc_ref[...])
        ot = jnp.where(ot >= 0.0, ot, 0.01 * ot)
        o_ref[...] = ot.T

    grid = pl.cdiv(n_nodes, bn)
    return pl.pallas_call(
        body,
        grid=(grid,),
        in_specs=[
            pl.BlockSpec((_NUM_WORKERS, 1, bn), lambda i: (0, 0, i)),
            pl.BlockSpec((1, bn), lambda i: (0, i)),
            pl.BlockSpec((1, bn), lambda i: (0, i)),
            pl.BlockSpec((out_c, 1), lambda i: (0, 0)),
            pl.BlockSpec((out_c, 1), lambda i: (0, 0)),
            pl.BlockSpec((out_c, 1), lambda i: (0, 0)),
        ],
        out_specs=pl.BlockSpec((bn, out_c), lambda i: (i, 0)),
        out_shape=jax.ShapeDtypeStruct((n_nodes, out_c), jnp.float32),
    )(accp, dis_row, x_row, w0c, w1c, bc)


def kernel(x, edge_index, edge_weight, W0, W1, b):
    n = x.shape[0]
    out_c = W0.shape[1]
    np_pad = ((n + 127) // 128) * 128
    ei = edge_index.astype(jnp.int32)
    ew = edge_weight.astype(jnp.float32)

    degp = _scatter_partials(ei, ew, np_pad)
    x_row = x.reshape(1, n)
    dis_row, s_row = _dis_s(degp, x_row, np_pad)
    dst_flat = ei[1]
    m = _gather_mul(ei, ew, s_row.reshape(np_pad))
    accp = _scatter_partials(dst_flat, m, np_pad)
    w0c = W0.astype(jnp.float32).reshape(out_c, 1)
    w1c = W1.astype(jnp.float32).reshape(out_c, 1)
    bc = b.astype(jnp.float32).reshape(out_c, 1)
    out = _tx1_dense(accp, dis_row, x_row, w0c, w1c, bc, n, out_c)
    return out


# speedup vs baseline: 1.1266x; 1.1256x over previous
import functools

import jax
import jax.numpy as jnp
from jax import lax
from jax.experimental import pallas as pl
from jax.experimental.pallas import tpu as pltpu
from jax.experimental.pallas import tpu_sc as plsc

_NUM_CORES = 2
_NUM_SUBCORES = 16
_NUM_WORKERS = _NUM_CORES * _NUM_SUBCORES
_LANES = 16
_CH = 2048


def _chunk_plan(e):
    assert e % _CH == 0, f"edge count {e} not divisible by {_CH}"
    nch_tot = e // _CH
    nfull = nch_tot // _NUM_WORKERS
    rem = nch_tot - nfull * _NUM_WORKERS
    assert nfull >= 2
    maxc = nfull + (1 if rem else 0)
    return nch_tot, nfull, rem, maxc


def _worker_id():
    return lax.axis_index("s") * _NUM_CORES + lax.axis_index("c")


def _scatter_partials(edge_index, values, np_pad):
    e = values.shape[0]
    packed = edge_index.ndim == 2
    nch_tot, nfull, rem, maxc = _chunk_plan(e)
    gp = _CH // _LANES
    ibuf_t = (pltpu.VMEM((2, _CH), jnp.int32) if packed
              else pltpu.VMEM((_CH,), jnp.int32))

    mesh = plsc.VectorSubcoreMesh(core_axis_name="c", subcore_axis_name="s")

    @functools.partial(
        pl.kernel,
        out_type=jax.ShapeDtypeStruct((_NUM_WORKERS, 1, np_pad), jnp.float32),
        mesh=mesh,
        compiler_params=pltpu.CompilerParams(needs_layout_passes=False),
        scratch_types=[
            pltpu.VMEM((np_pad,), jnp.float32),
            ibuf_t,
            ibuf_t,
            pltpu.VMEM((_CH,), jnp.float32),
            pltpu.VMEM((_CH,), jnp.float32),
            pltpu.SemaphoreType.DMA,
            pltpu.SemaphoreType.DMA,
            pltpu.SemaphoreType.DMA,
            pltpu.SemaphoreType.DMA,
        ],
    )
    def scatter_k(ei_hbm, val_hbm, out_hbm, acc, ib0, ib1, vb0, vb1,
                  si0, si1, sv0, sv1):
        wid = _worker_id()
        base_chunk = wid * nfull + jnp.minimum(wid, rem)
        count = nfull + jnp.where(wid < rem, 1, 0)

        zeros = jnp.zeros((_LANES,), jnp.float32)

        def zero_body(i, carry):
            acc[pl.ds(i * _LANES, _LANES)] = zeros
            return carry

        lax.fori_loop(0, np_pad // _LANES, zero_body, 0, unroll=8)

        def start(ci, ib, vb, si, sv):
            off = (base_chunk + ci) * _CH
            src = (ei_hbm.at[:, pl.ds(off, _CH)] if packed
                   else ei_hbm.at[pl.ds(off, _CH)])
            pltpu.async_copy(src, ib, si)
            pltpu.async_copy(val_hbm.at[pl.ds(off, _CH)], vb, sv)

        def wait(ib, vb, si, sv):
            src = (ei_hbm.at[:, pl.ds(0, _CH)] if packed
                   else ei_hbm.at[pl.ds(0, _CH)])
            pltpu.make_async_copy(src, ib, si).wait()
            pltpu.make_async_copy(val_hbm.at[pl.ds(0, _CH)], vb, sv).wait()

        def process(ib, vb):
            def body(j, carry):
                if packed:
                    idx = ib[1, pl.ds(j * _LANES, _LANES)]
                else:
                    idx = ib[pl.ds(j * _LANES, _LANES)]
                v = vb[pl.ds(j * _LANES, _LANES)]
                plsc.addupdate_scatter(acc, [idx], v)
                return carry

            lax.fori_loop(0, gp, body, 0, unroll=8)

        start(0, ib0, vb0, si0, sv0)
        start(1, ib1, vb1, si1, sv1)

        def pair(p, carry):
            ci0 = p * 2
            ci1 = ci0 + 1

            @pl.when(ci0 < count)
            def _():
                wait(ib0, vb0, si0, sv0)
                process(ib0, vb0)

            @pl.when(ci0 + 2 < count)
            def _():
                start(ci0 + 2, ib0, vb0, si0, sv0)

            @pl.when(ci1 < count)
            def _():
                wait(ib1, vb1, si1, sv1)
                process(ib1, vb1)

            @pl.when(ci1 + 2 < count)
            def _():
                start(ci1 + 2, ib1, vb1, si1, sv1)

            return carry

        lax.fori_loop(0, (maxc + 1) // 2, pair, 0)
        pltpu.sync_copy(acc, out_hbm.at[wid, 0, pl.ds(0, np_pad)])

    return scatter_k(edge_index, values)


def _gather_mul(edge_index, values, s_flat):
    e = values.shape[0]
    n_pad = s_flat.shape[0]
    nch_tot, nfull, rem, maxc = _chunk_plan(e)
    gp = _CH // _LANES

    mesh = plsc.VectorSubcoreMesh(core_axis_name="c", subcore_axis_name="s")

    @functools.partial(
        pl.kernel,
        out_type=jax.ShapeDtypeStruct((e,), jnp.float32),
        mesh=mesh,
        compiler_params=pltpu.CompilerParams(needs_layout_passes=False),
        scratch_types=[
            pltpu.VMEM((n_pad,), jnp.float32),
            pltpu.VMEM((_CH,), jnp.int32),
            pltpu.VMEM((_CH,), jnp.int32),
            pltpu.VMEM((_CH,), jnp.float32),
            pltpu.VMEM((_CH,), jnp.float32),
            pltpu.VMEM((_CH,), jnp.float32),
            pltpu.VMEM((_CH,), jnp.float32),
            pltpu.SemaphoreType.DMA,
            pltpu.SemaphoreType.DMA,
            pltpu.SemaphoreType.DMA,
            pltpu.SemaphoreType.DMA,
            pltpu.SemaphoreType.DMA,
            pltpu.SemaphoreType.DMA,
            pltpu.SemaphoreType.DMA,
        ],
    )
    def gather_k(ei_hbm, val_hbm, s_hbm, m_hbm, sv, ib0, ib1, wb0, wb1,
                 mb0, mb1, stab, si0, si1, sw0, sw1, so0, so1):
        wid = _worker_id()
        base_chunk = wid * nfull + jnp.minimum(wid, rem)
        count = nfull + jnp.where(wid < rem, 1, 0)

        pltpu.async_copy(s_hbm, sv, stab).wait()

        def start_in(ci, ib, wb, si, sw):
            off = (base_chunk + ci) * _CH
            pltpu.async_copy(ei_hbm.at[0, pl.ds(off, _CH)], ib, si)
            pltpu.async_copy(val_hbm.at[pl.ds(off, _CH)], wb, sw)

        def wait_in(ib, wb, si, sw):
            pltpu.make_async_copy(ei_hbm.at[0, pl.ds(0, _CH)], ib, si).wait()
            pltpu.make_async_copy(val_hbm.at[pl.ds(0, _CH)], wb, sw).wait()

        def wait_out(mb, so):
            pltpu.make_async_copy(mb, m_hbm.at[pl.ds(0, _CH)], so).wait()

        def process(ib, wb, mb):
            def body(j, carry):
                idx = ib[pl.ds(j * _LANES, _LANES)]
                w = wb[pl.ds(j * _LANES, _LANES)]
                vals = plsc.load_gather(sv, [idx])
                mb[pl.ds(j * _LANES, _LANES)] = vals * w
                return carry

            lax.fori_loop(0, gp, body, 0, unroll=8)

        def start_out(ci, mb, so):
            off = (base_chunk + ci) * _CH
            pltpu.async_copy(mb, m_hbm.at[pl.ds(off, _CH)], so)

        start_in(0, ib0, wb0, si0, sw0)
        start_in(1, ib1, wb1, si1, sw1)

        def pair(p, carry):
            ci0 = p * 2
            ci1 = ci0 + 1

            @pl.when(ci0 < count)
            def _():
                wait_in(ib0, wb0, si0, sw0)

                @pl.when(p > 0)
                def _():
                    wait_out(mb0, so0)

                process(ib0, wb0, mb0)
                start_out(ci0, mb0, so0)

            @pl.when(ci0 + 2 < count)
            def _():
                start_in(ci0 + 2, ib0, wb0, si0, sw0)

            @pl.when(ci1 < count)
            def _():
                wait_in(ib1, wb1, si1, sw1)

                @pl.when(p > 0)
                def _():
                    wait_out(mb1, so1)

                process(ib1, wb1, mb1)
                start_out(ci1, mb1, so1)

            @pl.when(ci1 + 2 < count)
            def _():
                start_in(ci1 + 2, ib1, wb1, si1, sw1)

            return carry

        lax.fori_loop(0, (maxc + 1) // 2, pair, 0)
        wait_out(mb0, so0)

        @pl.when(count > 1)
        def _():
            wait_out(mb1, so1)

    return gather_k(edge_index, values, s_flat)


def _dis_s(degp, x_row, np_pad, bn=8192):
    def body(degp_ref, x_ref, dis_ref, s_ref):
        d = jnp.sum(degp_ref[...], axis=0)
        dis = jnp.where(d > 0.0, lax.rsqrt(jnp.maximum(d, 1e-12)), 0.0)
        dis_ref[...] = dis
        s_ref[...] = dis * x_ref[...]

    grid = pl.cdiv(np_pad, bn)
    return pl.pallas_call(
        body,
        grid=(grid,),
        in_specs=[
            pl.BlockSpec((_NUM_WORKERS, 1, bn), lambda i: (0, 0, i)),
            pl.BlockSpec((1, bn), lambda i: (0, i)),
        ],
        out_specs=[
            pl.BlockSpec((1, bn), lambda i: (0, i)),
            pl.BlockSpec((1, bn), lambda i: (0, i)),
        ],
        out_shape=[
            jax.ShapeDtypeStruct((1, np_pad), jnp.float32),
            jax.ShapeDtypeStruct((1, np_pad), jnp.float32),
        ],
    )(degp, x_row)


def _tx1_dense(accp, dis_row, x_row, w0c, w1c, bc, n_nodes, out_c, bn=2048):
    def body(accp_ref, dis_ref, x_ref, w0c_ref, w1c_ref, bc_ref, o_ref):
        a = jnp.sum(accp_ref[...], axis=0)
        t = -dis_ref[...] * a
        ot = (x_ref[...] * w0c_ref[...] + t * w1c_ref[...]
              + bc_ref[...])
        ot = jnp.where(ot >= 0.0, ot, 0.01 * ot)
        o_ref[...] = ot

    grid = pl.cdiv(n_nodes, bn)
    return pl.pallas_call(
        body,
        grid=(grid,),
        in_specs=[
            pl.BlockSpec((_NUM_WORKERS, 1, bn), lambda i: (0, 0, i)),
            pl.BlockSpec((1, bn), lambda i: (0, i)),
            pl.BlockSpec((1, bn), lambda i: (0, i)),
            pl.BlockSpec((out_c, 1), lambda i: (0, 0)),
            pl.BlockSpec((out_c, 1), lambda i: (0, 0)),
            pl.BlockSpec((out_c, 1), lambda i: (0, 0)),
        ],
        out_specs=pl.BlockSpec((out_c, bn), lambda i: (0, i)),
        out_shape=jax.ShapeDtypeStruct((out_c, n_nodes), jnp.float32),
    )(accp, dis_row, x_row, w0c, w1c, bc)


def kernel(x, edge_index, edge_weight, W0, W1, b):
    n = x.shape[0]
    out_c = W0.shape[1]
    np_pad = ((n + 127) // 128) * 128
    ei = edge_index.astype(jnp.int32)
    ew = edge_weight.astype(jnp.float32)

    degp = _scatter_partials(ei, ew, np_pad)
    x_row = x.reshape(1, n)
    dis_row, s_row = _dis_s(degp, x_row, np_pad)
    m = _gather_mul(ei, ew, s_row.reshape(np_pad))
    accp = _scatter_partials(ei, m, np_pad)
    w0c = W0.astype(jnp.float32).reshape(out_c, 1)
    w1c = W1.astype(jnp.float32).reshape(out_c, 1)
    bc = b.astype(jnp.float32).reshape(out_c, 1)
    out_t = _tx1_dense(accp, dis_row, x_row, w0c, w1c, bc, n, out_c)
    return out_t.T


# speedup vs baseline: 1.1604x; 1.0300x over previous
import functools

import jax
import jax.numpy as jnp
from jax import lax
from jax.experimental import pallas as pl
from jax.experimental.pallas import tpu as pltpu
from jax.experimental.pallas import tpu_sc as plsc

_NUM_CORES = 2
_NUM_SUBCORES = 16
_NUM_WORKERS = _NUM_CORES * _NUM_SUBCORES
_LANES = 16
_CH = 2048


def _chunk_plan(e):
    assert e % _CH == 0, f"edge count {e} not divisible by {_CH}"
    nch_tot = e // _CH
    nfull = nch_tot // _NUM_WORKERS
    rem = nch_tot - nfull * _NUM_WORKERS
    assert nfull >= 2
    maxc = nfull + (1 if rem else 0)
    return nch_tot, nfull, rem, maxc


def _worker_id():
    return lax.axis_index("s") * _NUM_CORES + lax.axis_index("c")


def _scatter_partials(edge_index, values, np_pad):
    e = values.shape[0]
    packed = edge_index.ndim == 2
    nch_tot, nfull, rem, maxc = _chunk_plan(e)
    gp = _CH // _LANES
    ibuf_t = (pltpu.VMEM((2, _CH), jnp.int32) if packed
              else pltpu.VMEM((_CH,), jnp.int32))

    mesh = plsc.VectorSubcoreMesh(core_axis_name="c", subcore_axis_name="s")

    @functools.partial(
        pl.kernel,
        out_type=jax.ShapeDtypeStruct((_NUM_WORKERS, 1, np_pad), jnp.float32),
        mesh=mesh,
        compiler_params=pltpu.CompilerParams(needs_layout_passes=False),
        scratch_types=[
            pltpu.VMEM((np_pad,), jnp.float32),
            ibuf_t,
            ibuf_t,
            pltpu.VMEM((_CH,), jnp.float32),
            pltpu.VMEM((_CH,), jnp.float32),
            pltpu.SemaphoreType.DMA,
            pltpu.SemaphoreType.DMA,
            pltpu.SemaphoreType.DMA,
            pltpu.SemaphoreType.DMA,
        ],
    )
    def scatter_k(ei_hbm, val_hbm, out_hbm, acc, ib0, ib1, vb0, vb1,
                  si0, si1, sv0, sv1):
        wid = _worker_id()
        base_chunk = wid * nfull + jnp.minimum(wid, rem)
        count = nfull + jnp.where(wid < rem, 1, 0)

        zeros = jnp.zeros((_LANES,), jnp.float32)

        def zero_body(i, carry):
            acc[pl.ds(i * _LANES, _LANES)] = zeros
            return carry

        lax.fori_loop(0, np_pad // _LANES, zero_body, 0, unroll=8)

        def start(ci, ib, vb, si, sv):
            off = (base_chunk + ci) * _CH
            src = (ei_hbm.at[:, pl.ds(off, _CH)] if packed
                   else ei_hbm.at[pl.ds(off, _CH)])
            pltpu.async_copy(src, ib, si)
            pltpu.async_copy(val_hbm.at[pl.ds(off, _CH)], vb, sv)

        def wait(ib, vb, si, sv):
            src = (ei_hbm.at[:, pl.ds(0, _CH)] if packed
                   else ei_hbm.at[pl.ds(0, _CH)])
            pltpu.make_async_copy(src, ib, si).wait()
            pltpu.make_async_copy(val_hbm.at[pl.ds(0, _CH)], vb, sv).wait()

        def process(ib, vb):
            def body(j, carry):
                if packed:
                    idx = ib[1, pl.ds(j * _LANES, _LANES)]
                else:
                    idx = ib[pl.ds(j * _LANES, _LANES)]
                v = vb[pl.ds(j * _LANES, _LANES)]
                plsc.addupdate_scatter(acc, [idx], v)
                return carry

            lax.fori_loop(0, gp, body, 0, unroll=8)

        start(0, ib0, vb0, si0, sv0)
        start(1, ib1, vb1, si1, sv1)

        def pair(p, carry):
            ci0 = p * 2
            ci1 = ci0 + 1

            @pl.when(ci0 < count)
            def _():
                wait(ib0, vb0, si0, sv0)
                process(ib0, vb0)

            @pl.when(ci0 + 2 < count)
            def _():
                start(ci0 + 2, ib0, vb0, si0, sv0)

            @pl.when(ci1 < count)
            def _():
                wait(ib1, vb1, si1, sv1)
                process(ib1, vb1)

            @pl.when(ci1 + 2 < count)
            def _():
                start(ci1 + 2, ib1, vb1, si1, sv1)

            return carry

        lax.fori_loop(0, (maxc + 1) // 2, pair, 0)
        pltpu.sync_copy(acc, out_hbm.at[wid, 0, pl.ds(0, np_pad)])

    return scatter_k(edge_index, values)


def _gather_mul(edge_index, values, s_flat):
    e = values.shape[0]
    n_pad = s_flat.shape[0]
    nch_tot, nfull, rem, maxc = _chunk_plan(e)
    gp = _CH // _LANES

    mesh = plsc.VectorSubcoreMesh(core_axis_name="c", subcore_axis_name="s")

    @functools.partial(
        pl.kernel,
        out_type=jax.ShapeDtypeStruct((e,), jnp.float32),
        mesh=mesh,
        compiler_params=pltpu.CompilerParams(needs_layout_passes=False),
        scratch_types=[
            pltpu.VMEM((n_pad,), jnp.float32),
            pltpu.VMEM((_CH,), jnp.int32),
            pltpu.VMEM((_CH,), jnp.int32),
            pltpu.VMEM((_CH,), jnp.float32),
            pltpu.VMEM((_CH,), jnp.float32),
            pltpu.VMEM((_CH,), jnp.float32),
            pltpu.VMEM((_CH,), jnp.float32),
            pltpu.SemaphoreType.DMA,
            pltpu.SemaphoreType.DMA,
            pltpu.SemaphoreType.DMA,
            pltpu.SemaphoreType.DMA,
            pltpu.SemaphoreType.DMA,
            pltpu.SemaphoreType.DMA,
            pltpu.SemaphoreType.DMA,
        ],
    )
    def gather_k(ei_hbm, val_hbm, s_hbm, m_hbm, sv, ib0, ib1, wb0, wb1,
                 mb0, mb1, stab, si0, si1, sw0, sw1, so0, so1):
        wid = _worker_id()
        base_chunk = wid * nfull + jnp.minimum(wid, rem)
        count = nfull + jnp.where(wid < rem, 1, 0)

        rep = pltpu.async_copy(s_hbm, sv, stab)

        def start_in(ci, ib, wb, si, sw):
            off = (base_chunk + ci) * _CH
            pltpu.async_copy(ei_hbm.at[0, pl.ds(off, _CH)], ib, si)
            pltpu.async_copy(val_hbm.at[pl.ds(off, _CH)], wb, sw)

        def wait_in(ib, wb, si, sw):
            pltpu.make_async_copy(ei_hbm.at[0, pl.ds(0, _CH)], ib, si).wait()
            pltpu.make_async_copy(val_hbm.at[pl.ds(0, _CH)], wb, sw).wait()

        def wait_out(mb, so):
            pltpu.make_async_copy(mb, m_hbm.at[pl.ds(0, _CH)], so).wait()

        def process(ib, wb, mb):
            def body(j, carry):
                idx = ib[pl.ds(j * _LANES, _LANES)]
                w = wb[pl.ds(j * _LANES, _LANES)]
                vals = plsc.load_gather(sv, [idx])
                mb[pl.ds(j * _LANES, _LANES)] = vals * w
                return carry

            lax.fori_loop(0, gp, body, 0, unroll=8)

        def start_out(ci, mb, so):
            off = (base_chunk + ci) * _CH
            pltpu.async_copy(mb, m_hbm.at[pl.ds(off, _CH)], so)

        start_in(0, ib0, wb0, si0, sw0)
        start_in(1, ib1, wb1, si1, sw1)
        rep.wait()

        def pair(p, carry):
            ci0 = p * 2
            ci1 = ci0 + 1

            @pl.when(ci0 < count)
            def _():
                wait_in(ib0, wb0, si0, sw0)

                @pl.when(p > 0)
                def _():
                    wait_out(mb0, so0)

                process(ib0, wb0, mb0)
                start_out(ci0, mb0, so0)

            @pl.when(ci0 + 2 < count)
            def _():
                start_in(ci0 + 2, ib0, wb0, si0, sw0)

            @pl.when(ci1 < count)
            def _():
                wait_in(ib1, wb1, si1, sw1)

                @pl.when(p > 0)
                def _():
                    wait_out(mb1, so1)

                process(ib1, wb1, mb1)
                start_out(ci1, mb1, so1)

            @pl.when(ci1 + 2 < count)
            def _():
                start_in(ci1 + 2, ib1, wb1, si1, sw1)

            return carry

        lax.fori_loop(0, (maxc + 1) // 2, pair, 0)
        wait_out(mb0, so0)

        @pl.when(count > 1)
        def _():
            wait_out(mb1, so1)

    return gather_k(edge_index, values, s_flat)


def _dis_s(degp, x_row, np_pad, bn=8192):
    def body(degp_ref, x_ref, dis_ref, s_ref):
        d = jnp.sum(degp_ref[...], axis=0)
        dis = jnp.where(d > 0.0, lax.rsqrt(jnp.maximum(d, 1e-12)), 0.0)
        dis_ref[...] = dis
        s_ref[...] = dis * x_ref[...]

    grid = pl.cdiv(np_pad, bn)
    return pl.pallas_call(
        body,
        grid=(grid,),
        in_specs=[
            pl.BlockSpec((_NUM_WORKERS, 1, bn), lambda i: (0, 0, i)),
            pl.BlockSpec((1, bn), lambda i: (0, i)),
        ],
        out_specs=[
            pl.BlockSpec((1, bn), lambda i: (0, i)),
            pl.BlockSpec((1, bn), lambda i: (0, i)),
        ],
        out_shape=[
            jax.ShapeDtypeStruct((1, np_pad), jnp.float32),
            jax.ShapeDtypeStruct((1, np_pad), jnp.float32),
        ],
    )(degp, x_row)


def _tx1_dense(accp, dis_row, x_row, w0c, w1c, bc, n_nodes, out_c, bn=4096):
    def body(accp_ref, dis_ref, x_ref, w0c_ref, w1c_ref, bc_ref, o_ref):
        a = jnp.sum(accp_ref[...], axis=0)
        t = -dis_ref[...] * a
        ot = (x_ref[...] * w0c_ref[...] + t * w1c_ref[...]
              + bc_ref[...])
        ot = jnp.where(ot >= 0.0, ot, 0.01 * ot)
        o_ref[...] = ot

    grid = pl.cdiv(n_nodes, bn)
    return pl.pallas_call(
        body,
        grid=(grid,),
        in_specs=[
            pl.BlockSpec((_NUM_WORKERS, 1, bn), lambda i: (0, 0, i)),
            pl.BlockSpec((1, bn), lambda i: (0, i)),
            pl.BlockSpec((1, bn), lambda i: (0, i)),
            pl.BlockSpec((out_c, 1), lambda i: (0, 0)),
            pl.BlockSpec((out_c, 1), lambda i: (0, 0)),
            pl.BlockSpec((out_c, 1), lambda i: (0, 0)),
        ],
        out_specs=pl.BlockSpec((out_c, bn), lambda i: (0, i)),
        out_shape=jax.ShapeDtypeStruct((out_c, n_nodes), jnp.float32),
    )(accp, dis_row, x_row, w0c, w1c, bc)


def kernel(x, edge_index, edge_weight, W0, W1, b):
    n = x.shape[0]
    out_c = W0.shape[1]
    np_pad = ((n + 127) // 128) * 128
    ei = edge_index.astype(jnp.int32)
    ew = edge_weight.astype(jnp.float32)

    degp = _scatter_partials(ei, ew, np_pad)
    x_row = x.reshape(1, n)
    dis_row, s_row = _dis_s(degp, x_row, np_pad)
    m = _gather_mul(ei, ew, s_row.reshape(np_pad))
    accp = _scatter_partials(ei, m, np_pad)
    w0c = W0.astype(jnp.float32).reshape(out_c, 1)
    w1c = W1.astype(jnp.float32).reshape(out_c, 1)
    bc = b.astype(jnp.float32).reshape(out_c, 1)
    out_t = _tx1_dense(accp, dis_row, x_row, w0c, w1c, bc, n, out_c)
    return out_t.T
